# baseline (speedup 1.0000x reference)
import functools

import jax
import jax.numpy as jnp
from jax import lax
from jax.experimental import pallas as pl
from jax.experimental.pallas import tpu as pltpu
from jax.experimental.pallas import tpu_sc as plsc

N = 10000
E = 320000
D = 128
H = 128
DE = 16
NORM = 100.0

NC = 2
NS = 16
NW = NC * NS
CHUNK = 80
CROWS = E // CHUNK
NP = 10240
RPT = NP // NS
WB = 64
BE = 10000
BN = 1000

NCH_HI = 128
NCH_LO = 120
N_HI = 20
CB_LO = N_HI * NCH_HI


def _sigmoid(x):
    return 0.5 * jnp.tanh(0.5 * x) + 0.5


def _silu(x):
    return x * _sigmoid(x)


def _worker_chunks(wid):
    hi = wid < N_HI
    nch = jnp.where(hi, NCH_HI, NCH_LO)
    cbase = jnp.where(hi, wid * NCH_HI, CB_LO + (wid - N_HI) * NCH_LO)
    return nch, cbase


def _load_worker_indices(idx2d_hbm, dst, wid, cbase):
    pltpu.sync_copy(idx2d_hbm.at[pl.ds(cbase, NCH_LO)], dst.at[pl.ds(0, NCH_LO)])

    @pl.when(wid < N_HI)
    def _():
        pltpu.sync_copy(idx2d_hbm.at[pl.ds(cbase + NCH_LO, NCH_HI - NCH_LO)],
                        dst.at[pl.ds(NCH_LO, NCH_HI - NCH_LO)])


def _preproj_body(h_ref, ws_ref, wt_ref, p_ref, q_ref):
    hb = h_ref[...]
    p_ref[...] = jnp.dot(hb, ws_ref[...], preferred_element_type=jnp.float32)
    q_ref[...] = jnp.dot(hb, wt_ref[...], preferred_element_type=jnp.float32)


def _preproj(h, w_src, w_tgt):
    return pl.pallas_call(
        _preproj_body,
        grid=(N // BN,),
        in_specs=[
            pl.BlockSpec((BN, D), lambda i: (i, 0)),
            pl.BlockSpec((D, H), lambda i: (0, 0)),
            pl.BlockSpec((D, H), lambda i: (0, 0)),
        ],
        out_specs=[
            pl.BlockSpec((BN, H), lambda i: (i, 0)),
            pl.BlockSpec((BN, H), lambda i: (i, 0)),
        ],
        out_shape=[
            jax.ShapeDtypeStruct((N, H), jnp.float32),
            jax.ShapeDtypeStruct((N, H), jnp.float32),
        ],
    )(h, w_src, w_tgt)


def _sc_gather_body(p_hbm, q_hbm, row_hbm, col_hbm, g_hbm,
                    rid, cid, bufp0, bufq0, bufp1, bufq1, bufo0, bufo1,
                    semp0, semq0, semp1, semq1, semo0, semo1):
    wid = lax.axis_index("s") * NC + lax.axis_index("c")
    nch, cbase = _worker_chunks(wid)

    bufp = (bufp0, bufp1)
    bufq = (bufq0, bufq1)
    bufo = (bufo0, bufo1)
    semp = (semp0, semp1)
    semq = (semq0, semq1)
    semo = (semo0, semo1)

    _load_worker_indices(row_hbm, rid, wid, cbase)
    _load_worker_indices(col_hbm, cid, wid, cbase)

    def fire(i, b):
        pltpu.async_copy(p_hbm.at[rid.at[i]], bufp[b], semp[b])
        pltpu.async_copy(q_hbm.at[cid.at[i]], bufq[b], semq[b])

    def drain(i, b):
        pltpu.make_async_copy(p_hbm.at[rid.at[i]], bufp[b], semp[b]).wait()
        pltpu.make_async_copy(q_hbm.at[cid.at[i]], bufq[b], semq[b]).wait()

    def out_slice(i):
        return g_hbm.at[pl.ds((cbase + i) * CHUNK, CHUNK)]

    for b in (0, 1):
        fire(b, b)

    def pair(j, carry):
        for b in (0, 1):
            i = 2 * j + b
            drain(i, b)

            @pl.when(i >= 2)
            def _():
                pltpu.make_async_copy(bufo[b], out_slice(i - 2), semo[b]).wait()

            def addrow(rr, carry2, _b=b):
                for dr in range(4):
                    r = 4 * rr + dr
                    for k in range(H // 16):
                        sl = pl.ds(k * 16, 16)
                        bufo[_b][r, sl] = bufp[_b][r, sl] + bufq[_b][r, sl]
                return carry2

            lax.fori_loop(0, CHUNK // 4, addrow, 0)

            @pl.when(i + 2 < nch)
            def _():
                fire(i + 2, b)

            pltpu.async_copy(bufo[b], out_slice(i), semo[b])

        return carry

    lax.fori_loop(0, nch // 2, pair, 0)

    for b in (0, 1):
        pltpu.make_async_copy(bufo[b], out_slice(nch - 2 + b), semo[b]).wait()


def _sc_gather(p, q, row2d, col2d):
    call = pl.kernel(
        _sc_gather_body,
        out_type=jax.ShapeDtypeStruct((E, H), jnp.float32),
        mesh=plsc.VectorSubcoreMesh(core_axis_name="c", subcore_axis_name="s"),
        scratch_types=[
            pltpu.VMEM((NCH_HI, CHUNK), jnp.int32),
            pltpu.VMEM((NCH_HI, CHUNK), jnp.int32),
            pltpu.VMEM((CHUNK, H), jnp.float32),
            pltpu.VMEM((CHUNK, H), jnp.float32),
            pltpu.VMEM((CHUNK, H), jnp.float32),
            pltpu.VMEM((CHUNK, H), jnp.float32),
            pltpu.VMEM((CHUNK, H), jnp.float32),
            pltpu.VMEM((CHUNK, H), jnp.float32),
            pltpu.SemaphoreType.DMA,
            pltpu.SemaphoreType.DMA,
            pltpu.SemaphoreType.DMA,
            pltpu.SemaphoreType.DMA,
            pltpu.SemaphoreType.DMA,
            pltpu.SemaphoreType.DMA,
        ],
    )
    return call(p, q, row2d, col2d)


def _edge_mlp_body(g_ref, ea_ref, em_ref, wa1_ref, b1_ref,
                   w2_ref, b2_ref, watt_ref, batt_ref, mij_ref, ef_ref):
    pre = (g_ref[...]
           + jnp.dot(ea_ref[...], wa1_ref[...],
                     preferred_element_type=jnp.float32)
           + b1_ref[...])
    t1 = _silu(pre)
    m = _silu(jnp.dot(t1, w2_ref[...], preferred_element_type=jnp.float32)
              + b2_ref[...])
    att = _sigmoid(
        jnp.sum(m * watt_ref[...], axis=1, keepdims=True) + batt_ref[...])
    mij_ref[...] = m
    ef_ref[...] = m * (att * em_ref[...])


def _edge_mlp(g, edge_attr, edge_mask, w1a, b1e, w2e, b2e, wa_t, ba):
    return pl.pallas_call(
        _edge_mlp_body,
        grid=(E // BE,),
        in_specs=[
            pl.BlockSpec((BE, H), lambda i: (i, 0)),
            pl.BlockSpec((BE, DE), lambda i: (i, 0)),
            pl.BlockSpec((BE, 1), lambda i: (i, 0)),
            pl.BlockSpec((DE, H), lambda i: (0, 0)),
            pl.BlockSpec((1, H), lambda i: (0, 0)),
            pl.BlockSpec((H, H), lambda i: (0, 0)),
            pl.BlockSpec((1, H), lambda i: (0, 0)),
            pl.BlockSpec((1, H), lambda i: (0, 0)),
            pl.BlockSpec((1, 1), lambda i: (0, 0)),
        ],
        out_specs=[
            pl.BlockSpec((BE, H), lambda i: (i, 0)),
            pl.BlockSpec((BE, H), lambda i: (i, 0)),
        ],
        out_shape=[
            jax.ShapeDtypeStruct((E, H), jnp.float32),
            jax.ShapeDtypeStruct((E, H), jnp.float32),
        ],
    )(g, edge_attr, edge_mask, w1a, b1e, w2e, b2e, wa_t, ba)


def _sc_scatter_body(ef_hbm, row_hbm, out_hbm, rid, buf0, buf1, zbuf, acc,
                     sem0, sem1):
    c = lax.axis_index("c")
    s = lax.axis_index("s")
    wid = s * NC + c
    nch, cbase = _worker_chunks(wid)

    buf = (buf0, buf1)
    sem = (sem0, sem1)

    def zv(j, carry):
        r = j // (H // 16)
        k = j % (H // 16)
        zbuf[r, pl.ds(k * 16, 16)] = jnp.zeros((16,), jnp.float32)
        return carry

    lax.fori_loop(0, WB * (H // 16), zv, 0)

    def zc(j, carry):
        pltpu.sync_copy(zbuf, acc.at[pl.ds(s * RPT + j * WB, WB)])
        return carry

    lax.fori_loop(0, RPT // WB, zc, 0)

    _load_worker_indices(row_hbm, rid, wid, cbase)
    plsc.subcore_barrier()

    def fire(i, b):
        pltpu.async_copy(ef_hbm.at[pl.ds((cbase + i) * CHUNK, CHUNK)],
                         buf[b], sem[b])

    def drain(i, b):
        pltpu.make_async_copy(
            ef_hbm.at[pl.ds((cbase + i) * CHUNK, CHUNK)], buf[b],
            sem[b]).wait()

    for b in (0, 1):
        fire(b, b)

    def pair(j, carry):
        for b in (0, 1):
            i = 2 * j + b
            drain(i, b)
            pltpu.sync_copy(buf[b], acc.at[rid.at[i]], add=True)

            @pl.when(i + 2 < nch)
            def _():
                fire(i + 2, b)

        return carry

    lax.fori_loop(0, nch // 2, pair, 0)
    plsc.subcore_barrier()

    def wb(j, carry):
        r0 = s * RPT + j * WB
        pltpu.sync_copy(acc.at[pl.ds(r0, WB)], zbuf)
        pltpu.sync_copy(zbuf, out_hbm.at[c, pl.ds(r0, WB)])
        return carry

    lax.fori_loop(0, RPT // WB, wb, 0)


def _sc_scatter(ef, row2d):
    call = pl.kernel(
        _sc_scatter_body,
        out_type=jax.ShapeDtypeStruct((NC, NP, H), jnp.float32),
        mesh=plsc.VectorSubcoreMesh(core_axis_name="c", subcore_axis_name="s"),
        scratch_types=[
            pltpu.VMEM((NCH_HI, CHUNK), jnp.int32),
            pltpu.VMEM((CHUNK, H), jnp.float32),
            pltpu.VMEM((CHUNK, H), jnp.float32),
            pltpu.VMEM((WB, H), jnp.float32),
            pltpu.VMEM_SHARED((NP, H), jnp.float32),
            pltpu.SemaphoreType.DMA,
            pltpu.SemaphoreType.DMA,
        ],
    )
    return call(ef, row2d)


def _node_mlp_body(h_ref, p0_ref, p1_ref, nm_ref, w1h_ref, w1a_ref, b1_ref,
                   w2_ref, b2_ref, out_ref):
    hb = h_ref[...]
    agg = (p0_ref[...] + p1_ref[...]) * jnp.float32(1.0 / NORM)
    t = _silu(jnp.dot(hb, w1h_ref[...], preferred_element_type=jnp.float32)
              + jnp.dot(agg, w1a_ref[...], preferred_element_type=jnp.float32)
              + b1_ref[...])
    out_ref[...] = (hb
                    + jnp.dot(t, w2_ref[...],
                              preferred_element_type=jnp.float32)
                    + b2_ref[...]) * nm_ref[...]


def _node_mlp(h, partials, node_mask, w1h, w1a, b1n, w2n, b2n):
    nspec = pl.BlockSpec((BN, H), lambda i: (i, 0))
    return pl.pallas_call(
        _node_mlp_body,
        grid=(N // BN,),
        in_specs=[
            pl.BlockSpec((BN, D), lambda i: (i, 0)),
            nspec, nspec,
            pl.BlockSpec((BN, 1), lambda i: (i, 0)),
            pl.BlockSpec((D, H), lambda i: (0, 0)),
            pl.BlockSpec((H, H), lambda i: (0, 0)),
            pl.BlockSpec((1, H), lambda i: (0, 0)),
            pl.BlockSpec((H, D), lambda i: (0, 0)),
            pl.BlockSpec((1, D), lambda i: (0, 0)),
        ],
        out_specs=pl.BlockSpec((BN, D), lambda i: (i, 0)),
        out_shape=jax.ShapeDtypeStruct((N, D), jnp.float32),
    )(h, partials[0], partials[1], node_mask, w1h, w1a, b1n, w2n, b2n)


def kernel(h, edge_index, edge_attr, node_mask, edge_mask,
           W1e, b1e, W2e, b2e, Wa, ba, W1n, b1n, W2n, b2n):
    row2d = edge_index[0].reshape(CROWS, CHUNK)
    col2d = edge_index[1].reshape(CROWS, CHUNK)

    p, q = _preproj(h, W1e[:D], W1e[D:2 * D])
    g = _sc_gather(p, q, row2d, col2d)
    mij, ef = _edge_mlp(
        g, edge_attr, edge_mask,
        W1e[2 * D:], b1e.reshape(1, H), W2e, b2e.reshape(1, H),
        Wa.reshape(1, H), ba.reshape(1, 1))
    partials = _sc_scatter(ef, row2d)
    h_out = _node_mlp(
        h, partials, node_mask,
        W1n[:D], W1n[D:], b1n.reshape(1, H), W2n, b2n.reshape(1, D))
    return (h_out, mij)


# speedup vs baseline: 1.0019x; 1.0019x over previous
import functools

import jax
import jax.numpy as jnp
from jax import lax
from jax.experimental import pallas as pl
from jax.experimental.pallas import tpu as pltpu
from jax.experimental.pallas import tpu_sc as plsc

N = 10000
E = 320000
D = 128
H = 128
DE = 16
NORM = 100.0

NC = 2
NS = 16
NW = NC * NS
CHUNK = 80
CROWS = E // CHUNK
NP = 10240
RPT = NP // NS
WB = 64
BE = 10000
BN = 1000

NCH_HI = 128
NCH_LO = 120
N_HI = 20
CB_LO = N_HI * NCH_HI


def _sigmoid(x):
    return 0.5 * jnp.tanh(0.5 * x) + 0.5


def _silu(x):
    return x * _sigmoid(x)


def _worker_chunks(wid):
    hi = wid < N_HI
    nch = jnp.where(hi, NCH_HI, NCH_LO)
    cbase = jnp.where(hi, wid * NCH_HI, CB_LO + (wid - N_HI) * NCH_LO)
    return nch, cbase


def _load_worker_indices(idx2d_hbm, dst, wid, cbase):
    pltpu.sync_copy(idx2d_hbm.at[pl.ds(cbase, NCH_LO)], dst.at[pl.ds(0, NCH_LO)])

    @pl.when(wid < N_HI)
    def _():
        pltpu.sync_copy(idx2d_hbm.at[pl.ds(cbase + NCH_LO, NCH_HI - NCH_LO)],
                        dst.at[pl.ds(NCH_LO, NCH_HI - NCH_LO)])


def _preproj_body(h_ref, ws_ref, wt_ref, p_ref, q_ref):
    hb = h_ref[...]
    p_ref[...] = jnp.dot(hb, ws_ref[...], preferred_element_type=jnp.float32)
    q_ref[...] = jnp.dot(hb, wt_ref[...], preferred_element_type=jnp.float32)


def _preproj(h, w_src, w_tgt):
    return pl.pallas_call(
        _preproj_body,
        grid=(N // BN,),
        in_specs=[
            pl.BlockSpec((BN, D), lambda i: (i, 0)),
            pl.BlockSpec((D, H), lambda i: (0, 0)),
            pl.BlockSpec((D, H), lambda i: (0, 0)),
        ],
        out_specs=[
            pl.BlockSpec((BN, H), lambda i: (i, 0)),
            pl.BlockSpec((BN, H), lambda i: (i, 0)),
        ],
        out_shape=[
            jax.ShapeDtypeStruct((N, H), jnp.float32),
            jax.ShapeDtypeStruct((N, H), jnp.float32),
        ],
    )(h, w_src, w_tgt)


NSLOT = 4


def _sc_gather_body(p_hbm, q_hbm, row_hbm, col_hbm, g_hbm,
                    rid, cid,
                    bufp0, bufq0, bufp1, bufq1, bufp2, bufq2, bufp3, bufq3,
                    semp0, semq0, semp1, semq1, semp2, semq2, semp3, semq3):
    wid = lax.axis_index("s") * NC + lax.axis_index("c")
    nch, cbase = _worker_chunks(wid)

    bufp = (bufp0, bufp1, bufp2, bufp3)
    bufq = (bufq0, bufq1, bufq2, bufq3)
    semp = (semp0, semp1, semp2, semp3)
    semq = (semq0, semq1, semq2, semq3)

    _load_worker_indices(row_hbm, rid, wid, cbase)
    _load_worker_indices(col_hbm, cid, wid, cbase)

    def fire(i, b):
        pltpu.async_copy(p_hbm.at[rid.at[i]], bufp[b], semp[b])
        pltpu.async_copy(q_hbm.at[cid.at[i]], bufq[b], semq[b])

    def drain(i, b):
        pltpu.make_async_copy(p_hbm.at[rid.at[i]], bufp[b], semp[b]).wait()
        pltpu.make_async_copy(q_hbm.at[cid.at[i]], bufq[b], semq[b]).wait()

    for b in range(NSLOT):
        fire(b, b)

    def group(j, carry):
        for b in range(NSLOT):
            i = NSLOT * j + b
            drain(i, b)

            def addrow(rr, carry2, _b=b):
                for dr in range(4):
                    r = 4 * rr + dr
                    for k in range(H // 16):
                        sl = pl.ds(k * 16, 16)
                        bufp[_b][r, sl] = bufp[_b][r, sl] + bufq[_b][r, sl]
                return carry2

            lax.fori_loop(0, CHUNK // 4, addrow, 0)
            pltpu.sync_copy(bufp[b], g_hbm.at[pl.ds((cbase + i) * CHUNK, CHUNK)])

            @pl.when(i + NSLOT < nch)
            def _():
                fire(i + NSLOT, b)

        return carry

    lax.fori_loop(0, nch // NSLOT, group, 0)


def _sc_gather(p, q, row2d, col2d):
    call = pl.kernel(
        _sc_gather_body,
        out_type=jax.ShapeDtypeStruct((E, H), jnp.float32),
        mesh=plsc.VectorSubcoreMesh(core_axis_name="c", subcore_axis_name="s"),
        scratch_types=(
            [pltpu.VMEM((NCH_HI, CHUNK), jnp.int32)] * 2
            + [pltpu.VMEM((CHUNK, H), jnp.float32)] * (2 * NSLOT)
            + [pltpu.SemaphoreType.DMA] * (2 * NSLOT)
        ),
    )
    return call(p, q, row2d, col2d)


def _edge_mlp_body(g_ref, ea_ref, em_ref, wa1_ref, b1_ref,
                   w2_ref, b2_ref, watt_ref, batt_ref, mij_ref, ef_ref):
    pre = (g_ref[...]
           + jnp.dot(ea_ref[...], wa1_ref[...],
                     preferred_element_type=jnp.float32)
           + b1_ref[...])
    t1 = _silu(pre)
    m = _silu(jnp.dot(t1, w2_ref[...], preferred_element_type=jnp.float32)
              + b2_ref[...])
    att = _sigmoid(
        jnp.sum(m * watt_ref[...], axis=1, keepdims=True) + batt_ref[...])
    mij_ref[...] = m
    ef_ref[...] = m * (att * em_ref[...])


def _edge_mlp(g, edge_attr, edge_mask, w1a, b1e, w2e, b2e, wa_t, ba):
    return pl.pallas_call(
        _edge_mlp_body,
        grid=(E // BE,),
        in_specs=[
            pl.BlockSpec((BE, H), lambda i: (i, 0)),
            pl.BlockSpec((BE, DE), lambda i: (i, 0)),
            pl.BlockSpec((BE, 1), lambda i: (i, 0)),
            pl.BlockSpec((DE, H), lambda i: (0, 0)),
            pl.BlockSpec((1, H), lambda i: (0, 0)),
            pl.BlockSpec((H, H), lambda i: (0, 0)),
            pl.BlockSpec((1, H), lambda i: (0, 0)),
            pl.BlockSpec((1, H), lambda i: (0, 0)),
            pl.BlockSpec((1, 1), lambda i: (0, 0)),
        ],
        out_specs=[
            pl.BlockSpec((BE, H), lambda i: (i, 0)),
            pl.BlockSpec((BE, H), lambda i: (i, 0)),
        ],
        out_shape=[
            jax.ShapeDtypeStruct((E, H), jnp.float32),
            jax.ShapeDtypeStruct((E, H), jnp.float32),
        ],
    )(g, edge_attr, edge_mask, w1a, b1e, w2e, b2e, wa_t, ba)


def _sc_scatter_body(ef_hbm, row_hbm, out_hbm, rid, buf0, buf1, zbuf, acc,
                     sem0, sem1):
    c = lax.axis_index("c")
    s = lax.axis_index("s")
    wid = s * NC + c
    nch, cbase = _worker_chunks(wid)

    buf = (buf0, buf1)
    sem = (sem0, sem1)

    def zv(j, carry):
        r = j // (H // 16)
        k = j % (H // 16)
        zbuf[r, pl.ds(k * 16, 16)] = jnp.zeros((16,), jnp.float32)
        return carry

    lax.fori_loop(0, WB * (H // 16), zv, 0)

    def zc(j, carry):
        pltpu.sync_copy(zbuf, acc.at[pl.ds(s * RPT + j * WB, WB)])
        return carry

    lax.fori_loop(0, RPT // WB, zc, 0)

    _load_worker_indices(row_hbm, rid, wid, cbase)
    plsc.subcore_barrier()

    def fire(i, b):
        pltpu.async_copy(ef_hbm.at[pl.ds((cbase + i) * CHUNK, CHUNK)],
                         buf[b], sem[b])

    def drain(i, b):
        pltpu.make_async_copy(
            ef_hbm.at[pl.ds((cbase + i) * CHUNK, CHUNK)], buf[b],
            sem[b]).wait()

    for b in (0, 1):
        fire(b, b)

    def pair(j, carry):
        for b in (0, 1):
            i = 2 * j + b
            drain(i, b)
            pltpu.sync_copy(buf[b], acc.at[rid.at[i]], add=True)

            @pl.when(i + 2 < nch)
            def _():
                fire(i + 2, b)

        return carry

    lax.fori_loop(0, nch // 2, pair, 0)
    plsc.subcore_barrier()

    def wb(j, carry):
        r0 = s * RPT + j * WB
        pltpu.sync_copy(acc.at[pl.ds(r0, WB)], zbuf)
        pltpu.sync_copy(zbuf, out_hbm.at[c, pl.ds(r0, WB)])
        return carry

    lax.fori_loop(0, RPT // WB, wb, 0)


def _sc_scatter(ef, row2d):
    call = pl.kernel(
        _sc_scatter_body,
        out_type=jax.ShapeDtypeStruct((NC, NP, H), jnp.float32),
        mesh=plsc.VectorSubcoreMesh(core_axis_name="c", subcore_axis_name="s"),
        scratch_types=[
            pltpu.VMEM((NCH_HI, CHUNK), jnp.int32),
            pltpu.VMEM((CHUNK, H), jnp.float32),
            pltpu.VMEM((CHUNK, H), jnp.float32),
            pltpu.VMEM((WB, H), jnp.float32),
            pltpu.VMEM_SHARED((NP, H), jnp.float32),
            pltpu.SemaphoreType.DMA,
            pltpu.SemaphoreType.DMA,
        ],
    )
    return call(ef, row2d)


def _node_mlp_body(h_ref, p0_ref, p1_ref, nm_ref, w1h_ref, w1a_ref, b1_ref,
                   w2_ref, b2_ref, out_ref):
    hb = h_ref[...]
    agg = (p0_ref[...] + p1_ref[...]) * jnp.float32(1.0 / NORM)
    t = _silu(jnp.dot(hb, w1h_ref[...], preferred_element_type=jnp.float32)
              + jnp.dot(agg, w1a_ref[...], preferred_element_type=jnp.float32)
              + b1_ref[...])
    out_ref[...] = (hb
                    + jnp.dot(t, w2_ref[...],
                              preferred_element_type=jnp.float32)
                    + b2_ref[...]) * nm_ref[...]


def _node_mlp(h, partials, node_mask, w1h, w1a, b1n, w2n, b2n):
    nspec = pl.BlockSpec((BN, H), lambda i: (i, 0))
    return pl.pallas_call(
        _node_mlp_body,
        grid=(N // BN,),
        in_specs=[
            pl.BlockSpec((BN, D), lambda i: (i, 0)),
            nspec, nspec,
            pl.BlockSpec((BN, 1), lambda i: (i, 0)),
            pl.BlockSpec((D, H), lambda i: (0, 0)),
            pl.BlockSpec((H, H), lambda i: (0, 0)),
            pl.BlockSpec((1, H), lambda i: (0, 0)),
            pl.BlockSpec((H, D), lambda i: (0, 0)),
            pl.BlockSpec((1, D), lambda i: (0, 0)),
        ],
        out_specs=pl.BlockSpec((BN, D), lambda i: (i, 0)),
        out_shape=jax.ShapeDtypeStruct((N, D), jnp.float32),
    )(h, partials[0], partials[1], node_mask, w1h, w1a, b1n, w2n, b2n)


def kernel(h, edge_index, edge_attr, node_mask, edge_mask,
           W1e, b1e, W2e, b2e, Wa, ba, W1n, b1n, W2n, b2n):
    row2d = edge_index[0].reshape(CROWS, CHUNK)
    col2d = edge_index[1].reshape(CROWS, CHUNK)

    p, q = _preproj(h, W1e[:D], W1e[D:2 * D])
    g = _sc_gather(p, q, row2d, col2d)
    mij, ef = _edge_mlp(
        g, edge_attr, edge_mask,
        W1e[2 * D:], b1e.reshape(1, H), W2e, b2e.reshape(1, H),
        Wa.reshape(1, H), ba.reshape(1, 1))
    partials = _sc_scatter(ef, row2d)
    h_out = _node_mlp(
        h, partials, node_mask,
        W1n[:D], W1n[D:], b1n.reshape(1, H), W2n, b2n.reshape(1, D))
    return (h_out, mij)
